# baseline (speedup 1.0000x reference)
import functools

import jax
import jax.numpy as jnp
from jax import lax
from jax.experimental import pallas as pl
from jax.experimental.pallas import tpu as pltpu
from jax.experimental.pallas import tpu_sc as plsc

B, T, C, E, H, K = 2, 2048, 1024, 8, 4096, 2
N = B * T
TILE = 512
NT = 24
PADROWS = NT * TILE
HCG = 2048
NH = H // HCG

NC, NS, L = 2, 16, 16
NW = NC * NS
RPW = PADROWS // NW
GCH = 64
NCH = RPW // GCH
TPW = N // NW
CW = 32
CP = C // 2


def _cumsum0(a):
    n = a.shape[0]
    d = 1
    while d < n:
        a = a + jnp.concatenate(
            [jnp.zeros((d, a.shape[1]), a.dtype), a[: n - d]], axis=0)
        d *= 2
    return a


def _router_body(x_ref, wg_ref, bg_ref,
                 pos1_ref, pos2_ref, p1_ref, p2_ref, te_ref, valid_ref,
                 xp_ref):
    xb = x_ref[...].astype(jnp.bfloat16)
    rb = xb.astype(jnp.float32)
    blo = jax.lax.bitcast_convert_type(rb[:, :CP], jnp.int32)
    bhi = jax.lax.bitcast_convert_type(rb[:, CP:], jnp.int32)
    xp_ref[...] = jnp.bitwise_or(
        jax.lax.shift_right_logical(blo, 16),
        jnp.bitwise_and(bhi, jnp.int32(-65536)))
    z = jax.lax.dot_general(
        xb, wg_ref[...].astype(jnp.bfloat16),
        (((1,), (0,)), ((), ())),
        preferred_element_type=jnp.float32,
    ) + bg_ref[...]
    zmax = jnp.max(z, axis=-1, keepdims=True)
    ez = jnp.exp(z - zmax)
    s = ez / jnp.sum(ez, axis=-1, keepdims=True)
    lanes = jax.lax.broadcasted_iota(jnp.int32, (N, E), 1)
    m1 = jnp.max(s, axis=-1, keepdims=True)
    i1 = jnp.min(jnp.where(s == m1, lanes, E), axis=-1, keepdims=True)
    sm = jnp.where(lanes == i1, -jnp.inf, s)
    m2 = jnp.max(sm, axis=-1, keepdims=True)
    i2 = jnp.min(jnp.where(sm == m2, lanes, E), axis=-1, keepdims=True)
    e2 = jnp.exp(m2 - m1)
    p1_ref[...] = 1.0 / (1.0 + e2)
    p2_ref[...] = e2 / (1.0 + e2)

    flag = (jnp.equal(lanes, i1) | jnp.equal(lanes, i2)).astype(jnp.int32)
    cum = _cumsum0(flag)
    counts = cum[N - 1:N, :]
    tiles_e = (counts + (TILE - 1)) // TILE
    ct = tiles_e
    d = 1
    while d < E:
        ct = ct + jnp.concatenate(
            [jnp.zeros((1, d), ct.dtype), ct[:, : E - d]], axis=1)
        d *= 2
    base = (ct - tiles_e) * TILE
    dest = base + cum - 1
    pos1_ref[...] = jnp.sum(jnp.where(lanes == i1, dest, 0), axis=-1,
                            keepdims=True)
    pos2_ref[...] = jnp.sum(jnp.where(lanes == i2, dest, 0), axis=-1,
                            keepdims=True)

    jt = jax.lax.broadcasted_iota(jnp.int32, (NT, E), 0)
    ctb = jnp.broadcast_to(ct, (NT, E))
    te = jnp.sum((jt >= ctb).astype(jnp.int32), axis=-1, keepdims=True)
    te_ref[...] = jnp.minimum(te, E - 1)
    total = ctb[:, E - 1:E]
    valid_ref[...] = (jt[:, :1] < total).astype(jnp.int32)


def _router(x2d, Wg, bg):
    return pl.pallas_call(
        _router_body,
        out_shape=[
            jax.ShapeDtypeStruct((N, 1), jnp.int32),
            jax.ShapeDtypeStruct((N, 1), jnp.int32),
            jax.ShapeDtypeStruct((N, 1), jnp.float32),
            jax.ShapeDtypeStruct((N, 1), jnp.float32),
            jax.ShapeDtypeStruct((NT, 1), jnp.int32),
            jax.ShapeDtypeStruct((NT, 1), jnp.int32),
            jax.ShapeDtypeStruct((N, CP), jnp.int32),
        ],
    )(x2d, Wg, bg.reshape(1, E))


def _dispatch_body(pos1_hbm, pos2_hbm, pa_hbm, pb_hbm, x_hbm,
                   xs_hbm, scl_hbm,
                   pos1_v, pos2_v, pa_v, pb_v, rid2_v, scl_v,
                   rows_a, rows_b, sem_a, sem_b):
    wid = lax.axis_index("s") * NC + lax.axis_index("c")
    base = wid * RPW
    pltpu.sync_copy(pos1_hbm, pos1_v)
    pltpu.sync_copy(pos2_hbm, pos2_v)
    pltpu.sync_copy(pa_hbm, pa_v)
    pltpu.sync_copy(pb_hbm, pb_v)

    zf = jnp.zeros((L,), jnp.float32)
    zi = jnp.zeros((L,), jnp.int32)
    for i in range(RPW // L):
        scl_v[pl.ds(i * L, L)] = zf
    for c in range(NCH):
        for j in range(GCH // L):
            rid2_v[c, pl.ds(j * L, L)] = zi

    def scan(i, carry):
        tok = lax.iota(jnp.int32, L) + i * L
        for pv, sv in ((pos1_v, pa_v), (pos2_v, pb_v)):
            pos = pv[pl.ds(i * L, L)]
            rel = pos - base
            m = jnp.logical_and(rel >= 0, rel < RPW)
            relc = jnp.where(m, rel, 0)
            plsc.store_scatter(rid2_v, [lax.div(relc, GCH), lax.rem(relc, GCH)],
                               tok, mask=m)
            plsc.store_scatter(scl_v, [relc], sv[pl.ds(i * L, L)], mask=m)
        return carry

    lax.fori_loop(0, N // L, scan, 0)

    pltpu.sync_copy(scl_v, scl_hbm.at[pl.ds(base, RPW)])
    rows = (rows_a, rows_b)
    sems = (sem_a, sem_b)
    g = {}
    g[0] = pltpu.async_copy(x_hbm.at[rid2_v.at[0]], rows[0], sems[0])
    g[1] = pltpu.async_copy(x_hbm.at[rid2_v.at[1]], rows[1], sems[1])
    for c in range(NCH):
        b = c & 1
        g[c].wait()
        st = pltpu.async_copy(rows[b],
                              xs_hbm.at[pl.ds(base + c * GCH, GCH)],
                              sems[b])
        st.wait()
        if c + 2 < NCH:
            g[c + 2] = pltpu.async_copy(x_hbm.at[rid2_v.at[c + 2]],
                                        rows[b], sems[b])


def _dispatch(pos1, pos2, pa, pb, x2d):
    mesh = plsc.VectorSubcoreMesh(core_axis_name="c", subcore_axis_name="s")
    return pl.kernel(
        _dispatch_body,
        out_type=[
            jax.ShapeDtypeStruct((PADROWS, CP), jnp.int32),
            jax.ShapeDtypeStruct((PADROWS,), jnp.float32),
        ],
        mesh=mesh,
        scratch_types=[
            pltpu.VMEM((N,), jnp.int32),
            pltpu.VMEM((N,), jnp.int32),
            pltpu.VMEM((N,), jnp.float32),
            pltpu.VMEM((N,), jnp.float32),
            pltpu.VMEM((NCH, GCH), jnp.int32),
            pltpu.VMEM((RPW,), jnp.float32),
            pltpu.VMEM((GCH, CP), jnp.int32),
            pltpu.VMEM((GCH, CP), jnp.int32),
            pltpu.SemaphoreType.DMA,
            pltpu.SemaphoreType.DMA,
        ],
        compiler_params=pltpu.CompilerParams(needs_layout_passes=False),
    )(pos1, pos2, pa, pb, x2d)


def _gffn_body(te_ref, valid_ref, xs_ref, w1_ref, b1_ref, w2_ref, b2_ref,
               scl_ref, out_ref, acc, xbf):
    t = pl.program_id(0)
    h = pl.program_id(1)

    @pl.when(valid_ref[t] == 1)
    def _():
        @pl.when(h == 0)
        def _():
            w = xs_ref[...]
            lo = jax.lax.bitcast_convert_type(
                jax.lax.shift_left(w, 16), jnp.float32)
            hi = jax.lax.bitcast_convert_type(
                jnp.bitwise_and(w, jnp.int32(-65536)), jnp.float32)
            xbf[...] = jnp.concatenate([lo, hi], axis=1).astype(jnp.bfloat16)
            acc[...] = jnp.zeros((TILE, C), jnp.float32) + b2_ref[0]

        hh = jax.lax.dot_general(
            xbf[...], w1_ref[0], (((1,), (0,)), ((), ())),
            preferred_element_type=jnp.float32,
        ) + b1_ref[0]
        hbf = jnp.maximum(hh, 0.0).astype(jnp.bfloat16)
        acc[...] += jax.lax.dot_general(
            hbf, w2_ref[0], (((1,), (0,)), ((), ())),
            preferred_element_type=jnp.float32,
        )

        @pl.when(h == NH - 1)
        def _():
            rb = ((acc[...] * scl_ref[...]).astype(jnp.bfloat16)
                  .astype(jnp.float32))
            blo = jax.lax.bitcast_convert_type(rb[:, :CP], jnp.int32)
            bhi = jax.lax.bitcast_convert_type(rb[:, CP:], jnp.int32)
            out_ref[...] = jnp.bitwise_or(
                jax.lax.shift_right_logical(blo, 16),
                jnp.bitwise_and(bhi, jnp.int32(-65536)))


def _grouped_ffn(te, valid, xs, W1bf, b1r, W2bf, b2r, scl):
    def hh_of(t, h):
        return jnp.where(t % 2 == 1, NH - 1 - h, h)

    grid_spec = pltpu.PrefetchScalarGridSpec(
        num_scalar_prefetch=2,
        grid=(NT, NH),
        in_specs=[
            pl.BlockSpec((TILE, CP), lambda t, h, te, va: (t, 0)),
            pl.BlockSpec((1, C, HCG), lambda t, h, te, va: (te[t], 0, hh_of(t, h))),
            pl.BlockSpec((1, 1, HCG), lambda t, h, te, va: (te[t], 0, hh_of(t, h))),
            pl.BlockSpec((1, HCG, C), lambda t, h, te, va: (te[t], hh_of(t, h), 0)),
            pl.BlockSpec((1, 1, C), lambda t, h, te, va: (te[t], 0, 0)),
            pl.BlockSpec((TILE, 1), lambda t, h, te, va: (t, 0)),
        ],
        out_specs=pl.BlockSpec((TILE, CP), lambda t, h, te, va: (t, 0)),
        scratch_shapes=[pltpu.VMEM((TILE, C), jnp.float32),
                        pltpu.VMEM((TILE, C), jnp.bfloat16)],
    )
    return pl.pallas_call(
        _gffn_body,
        grid_spec=grid_spec,
        out_shape=jax.ShapeDtypeStruct((PADROWS, CP), jnp.int32),
        compiler_params=pltpu.CompilerParams(
            dimension_semantics=("arbitrary", "arbitrary"),
        ),
    )(te, valid, xs, W1bf, b1r, W2bf, b2r, scl)


def _combine_body(pos1_hbm, pos2_hbm, ys_hbm, out_hbm,
                  p1b, p2b, rows_a, rows_b, sem):
    wid = lax.axis_index("s") * NC + lax.axis_index("c")
    tbase = wid * TPW
    for c in range(TPW // CW):
        pltpu.sync_copy(pos1_hbm.at[pl.ds(tbase + c * CW, CW)], p1b.at[c])
        pltpu.sync_copy(pos2_hbm.at[pl.ds(tbase + c * CW, CW)], p2b.at[c])
        pltpu.async_copy(ys_hbm.at[p1b.at[c]], rows_a, sem).wait()
        pltpu.async_copy(ys_hbm.at[p2b.at[c]], rows_b, sem).wait()

        def add_chunk(l, carry):
            lo = l * L
            for r in range(CW):
                a = plsc.bitcast(rows_a[r, pl.ds(lo, L)], jnp.bfloat16)
                b = plsc.bitcast(rows_b[r, pl.ds(lo, L)], jnp.bfloat16)
                rows_a[r, pl.ds(lo, L)] = plsc.bitcast(a + b, jnp.int32)
            return carry

        lax.fori_loop(0, CP // L, add_chunk, 0)
        pltpu.sync_copy(rows_a, out_hbm.at[pl.ds(tbase + c * CW, CW)])


def _combine(pos1, pos2, ys):
    mesh = plsc.VectorSubcoreMesh(core_axis_name="c", subcore_axis_name="s")
    return pl.kernel(
        _combine_body,
        out_type=jax.ShapeDtypeStruct((N, CP), jnp.int32),
        mesh=mesh,
        scratch_types=[
            pltpu.VMEM((TPW // CW, CW), jnp.int32),
            pltpu.VMEM((TPW // CW, CW), jnp.int32),
            pltpu.VMEM((CW, CP), jnp.int32),
            pltpu.VMEM((CW, CP), jnp.int32),
            pltpu.SemaphoreType.DMA,
        ],
        compiler_params=pltpu.CompilerParams(needs_layout_passes=False),
    )(pos1, pos2, ys)


@jax.jit
def kernel(x, Wg, bg, W1, b1, W2, b2):
    x2d = x.reshape(N, C)
    pos1, pos2, p1, p2, te, valid, xp = _router(x2d, Wg, bg)
    xs32, scl = _dispatch(pos1.reshape(N), pos2.reshape(N),
                          p1.reshape(N), p2.reshape(N), xp)
    ys32 = _grouped_ffn(te.reshape(NT), valid.reshape(NT), xs32,
                        W1.astype(jnp.bfloat16), b1.reshape(E, 1, H),
                        W2.astype(jnp.bfloat16), b2.reshape(E, 1, C),
                        scl.reshape(PADROWS, 1))
    out32 = _combine(pos1.reshape(N), pos2.reshape(N), ys32)
    lo = jax.lax.bitcast_convert_type(
        jax.lax.shift_left(out32, 16), jnp.float32)
    hi = jax.lax.bitcast_convert_type(
        jnp.bitwise_and(out32, jnp.int32(-65536)), jnp.float32)
    out2d = jnp.concatenate([lo, hi], axis=1)
    return out2d.reshape(B, T, C)


# speedup vs baseline: 1.1135x; 1.1135x over previous
import functools

import jax
import jax.numpy as jnp
from jax import lax
from jax.experimental import pallas as pl
from jax.experimental.pallas import tpu as pltpu
from jax.experimental.pallas import tpu_sc as plsc

B, T, C, E, H, K = 2, 2048, 1024, 8, 4096, 2
N = B * T
TILE = 256
NT = 40
PADROWS = NT * TILE
HCG = 2048
NH = H // HCG

NC, NS, L = 2, 16, 16
NW = NC * NS
RPW = PADROWS // NW
GCH = 80
NCH = RPW // GCH
TPW = N // NW
CW = 64
CP = C // 2


def _cumsum0(a):
    n = a.shape[0]
    d = 1
    while d < n:
        a = a + jnp.concatenate(
            [jnp.zeros((d, a.shape[1]), a.dtype), a[: n - d]], axis=0)
        d *= 2
    return a


def _router_body(x_ref, wg_ref, bg_ref,
                 pos1_ref, pos2_ref, p1_ref, p2_ref, te_ref, valid_ref,
                 xp_ref):
    xb = x_ref[...].astype(jnp.bfloat16)
    rb = xb.astype(jnp.float32)
    blo = jax.lax.bitcast_convert_type(rb[:, :CP], jnp.int32)
    bhi = jax.lax.bitcast_convert_type(rb[:, CP:], jnp.int32)
    xp_ref[...] = jnp.bitwise_or(
        jax.lax.shift_right_logical(blo, 16),
        jnp.bitwise_and(bhi, jnp.int32(-65536)))
    z = jax.lax.dot_general(
        xb, wg_ref[...].astype(jnp.bfloat16),
        (((1,), (0,)), ((), ())),
        preferred_element_type=jnp.float32,
    ) + bg_ref[...]
    zmax = jnp.max(z, axis=-1, keepdims=True)
    ez = jnp.exp(z - zmax)
    s = ez / jnp.sum(ez, axis=-1, keepdims=True)
    lanes = jax.lax.broadcasted_iota(jnp.int32, (N, E), 1)
    m1 = jnp.max(s, axis=-1, keepdims=True)
    i1 = jnp.min(jnp.where(s == m1, lanes, E), axis=-1, keepdims=True)
    sm = jnp.where(lanes == i1, -jnp.inf, s)
    m2 = jnp.max(sm, axis=-1, keepdims=True)
    i2 = jnp.min(jnp.where(sm == m2, lanes, E), axis=-1, keepdims=True)
    e2 = jnp.exp(m2 - m1)
    p1_ref[...] = 1.0 / (1.0 + e2)
    p2_ref[...] = e2 / (1.0 + e2)

    flag = (jnp.equal(lanes, i1) | jnp.equal(lanes, i2)).astype(jnp.int32)
    cum = _cumsum0(flag)
    counts = cum[N - 1:N, :]
    tiles_e = (counts + (TILE - 1)) // TILE
    ct = tiles_e
    d = 1
    while d < E:
        ct = ct + jnp.concatenate(
            [jnp.zeros((1, d), ct.dtype), ct[:, : E - d]], axis=1)
        d *= 2
    base = (ct - tiles_e) * TILE
    dest = base + cum - 1
    pos1_ref[...] = jnp.sum(jnp.where(lanes == i1, dest, 0), axis=-1,
                            keepdims=True)
    pos2_ref[...] = jnp.sum(jnp.where(lanes == i2, dest, 0), axis=-1,
                            keepdims=True)

    jt = jax.lax.broadcasted_iota(jnp.int32, (NT, E), 0)
    ctb = jnp.broadcast_to(ct, (NT, E))
    te = jnp.sum((jt >= ctb).astype(jnp.int32), axis=-1, keepdims=True)
    te_ref[...] = jnp.minimum(te, E - 1)
    total = ctb[:, E - 1:E]
    valid_ref[...] = (jt[:, :1] < total).astype(jnp.int32)


def _router(x2d, Wg, bg):
    return pl.pallas_call(
        _router_body,
        out_shape=[
            jax.ShapeDtypeStruct((N, 1), jnp.int32),
            jax.ShapeDtypeStruct((N, 1), jnp.int32),
            jax.ShapeDtypeStruct((N, 1), jnp.float32),
            jax.ShapeDtypeStruct((N, 1), jnp.float32),
            jax.ShapeDtypeStruct((NT, 1), jnp.int32),
            jax.ShapeDtypeStruct((NT, 1), jnp.int32),
            jax.ShapeDtypeStruct((N, CP), jnp.int32),
        ],
    )(x2d, Wg, bg.reshape(1, E))


def _dispatch_body(pos1_hbm, pos2_hbm, pa_hbm, pb_hbm, x_hbm,
                   xs_hbm, scl_hbm,
                   pos1_v, pos2_v, pa_v, pb_v, rid2_v, scl_v,
                   rows_a, rows_b, sem_a, sem_b):
    wid = lax.axis_index("s") * NC + lax.axis_index("c")
    base = wid * RPW
    pltpu.sync_copy(pos1_hbm, pos1_v)
    pltpu.sync_copy(pos2_hbm, pos2_v)
    pltpu.sync_copy(pa_hbm, pa_v)
    pltpu.sync_copy(pb_hbm, pb_v)

    zf = jnp.zeros((L,), jnp.float32)
    zi = jnp.zeros((L,), jnp.int32)
    for i in range(RPW // L):
        scl_v[pl.ds(i * L, L)] = zf
    for c in range(NCH):
        for j in range(GCH // L):
            rid2_v[c, pl.ds(j * L, L)] = zi

    def scan(i, carry):
        tok = lax.iota(jnp.int32, L) + i * L
        for pv, sv in ((pos1_v, pa_v), (pos2_v, pb_v)):
            pos = pv[pl.ds(i * L, L)]
            rel = pos - base
            m = jnp.logical_and(rel >= 0, rel < RPW)
            relc = jnp.where(m, rel, 0)
            plsc.store_scatter(rid2_v, [lax.div(relc, GCH), lax.rem(relc, GCH)],
                               tok, mask=m)
            plsc.store_scatter(scl_v, [relc], sv[pl.ds(i * L, L)], mask=m)
        return carry

    lax.fori_loop(0, N // L, scan, 0)

    pltpu.sync_copy(scl_v, scl_hbm.at[pl.ds(base, RPW)])
    rows = (rows_a, rows_b)
    sems = (sem_a, sem_b)
    g = {}
    g[0] = pltpu.async_copy(x_hbm.at[rid2_v.at[0]], rows[0], sems[0])
    g[1] = pltpu.async_copy(x_hbm.at[rid2_v.at[1]], rows[1], sems[1])
    for c in range(NCH):
        b = c & 1
        g[c].wait()
        st = pltpu.async_copy(rows[b],
                              xs_hbm.at[pl.ds(base + c * GCH, GCH)],
                              sems[b])
        st.wait()
        if c + 2 < NCH:
            g[c + 2] = pltpu.async_copy(x_hbm.at[rid2_v.at[c + 2]],
                                        rows[b], sems[b])


def _dispatch(pos1, pos2, pa, pb, x2d):
    mesh = plsc.VectorSubcoreMesh(core_axis_name="c", subcore_axis_name="s")
    return pl.kernel(
        _dispatch_body,
        out_type=[
            jax.ShapeDtypeStruct((PADROWS, CP), jnp.int32),
            jax.ShapeDtypeStruct((PADROWS,), jnp.float32),
        ],
        mesh=mesh,
        scratch_types=[
            pltpu.VMEM((N,), jnp.int32),
            pltpu.VMEM((N,), jnp.int32),
            pltpu.VMEM((N,), jnp.float32),
            pltpu.VMEM((N,), jnp.float32),
            pltpu.VMEM((NCH, GCH), jnp.int32),
            pltpu.VMEM((RPW,), jnp.float32),
            pltpu.VMEM((GCH, CP), jnp.int32),
            pltpu.VMEM((GCH, CP), jnp.int32),
            pltpu.SemaphoreType.DMA,
            pltpu.SemaphoreType.DMA,
        ],
        compiler_params=pltpu.CompilerParams(needs_layout_passes=False),
    )(pos1, pos2, pa, pb, x2d)


def _gffn_body(te_ref, valid_ref, xs_ref, w1_ref, b1_ref, w2_ref, b2_ref,
               scl_ref, out_ref, acc, xbf):
    t = pl.program_id(0)
    h = pl.program_id(1)

    @pl.when(valid_ref[t] == 1)
    def _():
        @pl.when(h == 0)
        def _():
            w = xs_ref[...]
            lo = jax.lax.bitcast_convert_type(
                jax.lax.shift_left(w, 16), jnp.float32)
            hi = jax.lax.bitcast_convert_type(
                jnp.bitwise_and(w, jnp.int32(-65536)), jnp.float32)
            xbf[...] = jnp.concatenate([lo, hi], axis=1).astype(jnp.bfloat16)
            acc[...] = jnp.zeros((TILE, C), jnp.float32) + b2_ref[0]

        hh = jax.lax.dot_general(
            xbf[...], w1_ref[0], (((1,), (0,)), ((), ())),
            preferred_element_type=jnp.float32,
        ) + b1_ref[0]
        hbf = jnp.maximum(hh, 0.0).astype(jnp.bfloat16)
        acc[...] += jax.lax.dot_general(
            hbf, w2_ref[0], (((1,), (0,)), ((), ())),
            preferred_element_type=jnp.float32,
        )

        @pl.when(h == NH - 1)
        def _():
            rb = ((acc[...] * scl_ref[...]).astype(jnp.bfloat16)
                  .astype(jnp.float32))
            blo = jax.lax.bitcast_convert_type(rb[:, :CP], jnp.int32)
            bhi = jax.lax.bitcast_convert_type(rb[:, CP:], jnp.int32)
            out_ref[...] = jnp.bitwise_or(
                jax.lax.shift_right_logical(blo, 16),
                jnp.bitwise_and(bhi, jnp.int32(-65536)))


def _grouped_ffn(te, valid, xs, W1bf, b1r, W2bf, b2r, scl):
    def hh_of(t, h):
        return jnp.where(t % 2 == 1, NH - 1 - h, h)

    grid_spec = pltpu.PrefetchScalarGridSpec(
        num_scalar_prefetch=2,
        grid=(NT, NH),
        in_specs=[
            pl.BlockSpec((TILE, CP), lambda t, h, te, va: (t, 0)),
            pl.BlockSpec((1, C, HCG), lambda t, h, te, va: (te[t], 0, hh_of(t, h))),
            pl.BlockSpec((1, 1, HCG), lambda t, h, te, va: (te[t], 0, hh_of(t, h))),
            pl.BlockSpec((1, HCG, C), lambda t, h, te, va: (te[t], hh_of(t, h), 0)),
            pl.BlockSpec((1, 1, C), lambda t, h, te, va: (te[t], 0, 0)),
            pl.BlockSpec((TILE, 1), lambda t, h, te, va: (t, 0)),
        ],
        out_specs=pl.BlockSpec((TILE, CP), lambda t, h, te, va: (t, 0)),
        scratch_shapes=[pltpu.VMEM((TILE, C), jnp.float32),
                        pltpu.VMEM((TILE, C), jnp.bfloat16)],
    )
    return pl.pallas_call(
        _gffn_body,
        grid_spec=grid_spec,
        out_shape=jax.ShapeDtypeStruct((PADROWS, CP), jnp.int32),
        compiler_params=pltpu.CompilerParams(
            dimension_semantics=("arbitrary", "arbitrary"),
        ),
    )(te, valid, xs, W1bf, b1r, W2bf, b2r, scl)


def _combine_body(pos1_hbm, pos2_hbm, ys_hbm, out_hbm,
                  p1b, p2b, rows_a, rows_b, sem):
    wid = lax.axis_index("s") * NC + lax.axis_index("c")
    tbase = wid * TPW
    for c in range(TPW // CW):
        pltpu.sync_copy(pos1_hbm.at[pl.ds(tbase + c * CW, CW)], p1b.at[c])
        pltpu.sync_copy(pos2_hbm.at[pl.ds(tbase + c * CW, CW)], p2b.at[c])
        pltpu.async_copy(ys_hbm.at[p1b.at[c]], rows_a, sem).wait()
        pltpu.async_copy(ys_hbm.at[p2b.at[c]], rows_b, sem).wait()

        def add_chunk(l, carry):
            lo = l * L
            for r in range(CW):
                a = plsc.bitcast(rows_a[r, pl.ds(lo, L)], jnp.bfloat16)
                b = plsc.bitcast(rows_b[r, pl.ds(lo, L)], jnp.bfloat16)
                rows_a[r, pl.ds(lo, L)] = plsc.bitcast(a + b, jnp.int32)
            return carry

        lax.fori_loop(0, CP // L, add_chunk, 0)
        pltpu.sync_copy(rows_a, out_hbm.at[pl.ds(tbase + c * CW, CW)])


def _combine(pos1, pos2, ys):
    mesh = plsc.VectorSubcoreMesh(core_axis_name="c", subcore_axis_name="s")
    return pl.kernel(
        _combine_body,
        out_type=jax.ShapeDtypeStruct((N, CP), jnp.int32),
        mesh=mesh,
        scratch_types=[
            pltpu.VMEM((TPW // CW, CW), jnp.int32),
            pltpu.VMEM((TPW // CW, CW), jnp.int32),
            pltpu.VMEM((CW, CP), jnp.int32),
            pltpu.VMEM((CW, CP), jnp.int32),
            pltpu.SemaphoreType.DMA,
        ],
        compiler_params=pltpu.CompilerParams(needs_layout_passes=False),
    )(pos1, pos2, ys)


@jax.jit
def kernel(x, Wg, bg, W1, b1, W2, b2):
    x2d = x.reshape(N, C)
    pos1, pos2, p1, p2, te, valid, xp = _router(x2d, Wg, bg)
    xs32, scl = _dispatch(pos1.reshape(N), pos2.reshape(N),
                          p1.reshape(N), p2.reshape(N), xp)
    ys32 = _grouped_ffn(te.reshape(NT), valid.reshape(NT), xs32,
                        W1.astype(jnp.bfloat16), b1.reshape(E, 1, H),
                        W2.astype(jnp.bfloat16), b2.reshape(E, 1, C),
                        scl.reshape(PADROWS, 1))
    out32 = _combine(pos1.reshape(N), pos2.reshape(N), ys32)
    lo = jax.lax.bitcast_convert_type(
        jax.lax.shift_left(out32, 16), jnp.float32)
    hi = jax.lax.bitcast_convert_type(
        jnp.bitwise_and(out32, jnp.int32(-65536)), jnp.float32)
    out2d = jnp.concatenate([lo, hi], axis=1)
    return out2d.reshape(B, T, C)


# speedup vs baseline: 1.2276x; 1.1025x over previous
import functools

import jax
import jax.numpy as jnp
from jax import lax
from jax.experimental import pallas as pl
from jax.experimental.pallas import tpu as pltpu
from jax.experimental.pallas import tpu_sc as plsc

B, T, C, E, H, K = 2, 2048, 1024, 8, 4096, 2
N = B * T
TILE = 256
NT = 40
PADROWS = NT * TILE
HCG = 4096
NH = H // HCG

NC, NS, L = 2, 16, 16
NW = NC * NS
RPW = PADROWS // NW
GCH = 80
NCH = RPW // GCH
TPW = N // NW
CW = 32
CP = C // 2


def _cumsum0(a):
    n = a.shape[0]
    d = 1
    while d < n:
        a = a + jnp.concatenate(
            [jnp.zeros((d, a.shape[1]), a.dtype), a[: n - d]], axis=0)
        d *= 2
    return a


def _router_body(x_ref, wg_ref, bg_ref,
                 pos1_ref, pos2_ref, p1_ref, p2_ref, te_ref, valid_ref,
                 xp_ref):
    xb = x_ref[...].astype(jnp.bfloat16)
    rb = xb.astype(jnp.float32)
    blo = jax.lax.bitcast_convert_type(rb[:, :CP], jnp.int32)
    bhi = jax.lax.bitcast_convert_type(rb[:, CP:], jnp.int32)
    xp_ref[...] = jnp.bitwise_or(
        jax.lax.shift_right_logical(blo, 16),
        jnp.bitwise_and(bhi, jnp.int32(-65536)))
    z = jax.lax.dot_general(
        xb, wg_ref[...].astype(jnp.bfloat16),
        (((1,), (0,)), ((), ())),
        preferred_element_type=jnp.float32,
    ) + bg_ref[...]
    zmax = jnp.max(z, axis=-1, keepdims=True)
    ez = jnp.exp(z - zmax)
    s = ez / jnp.sum(ez, axis=-1, keepdims=True)
    lanes = jax.lax.broadcasted_iota(jnp.int32, (N, E), 1)
    m1 = jnp.max(s, axis=-1, keepdims=True)
    i1 = jnp.min(jnp.where(s == m1, lanes, E), axis=-1, keepdims=True)
    sm = jnp.where(lanes == i1, -jnp.inf, s)
    m2 = jnp.max(sm, axis=-1, keepdims=True)
    i2 = jnp.min(jnp.where(sm == m2, lanes, E), axis=-1, keepdims=True)
    e2 = jnp.exp(m2 - m1)
    p1_ref[...] = 1.0 / (1.0 + e2)
    p2_ref[...] = e2 / (1.0 + e2)

    flag = (jnp.equal(lanes, i1) | jnp.equal(lanes, i2)).astype(jnp.int32)
    cum = _cumsum0(flag)
    counts = cum[N - 1:N, :]
    tiles_e = (counts + (TILE - 1)) // TILE
    ct = tiles_e
    d = 1
    while d < E:
        ct = ct + jnp.concatenate(
            [jnp.zeros((1, d), ct.dtype), ct[:, : E - d]], axis=1)
        d *= 2
    base = (ct - tiles_e) * TILE
    dest = base + cum - 1
    pos1_ref[...] = jnp.sum(jnp.where(lanes == i1, dest, 0), axis=-1,
                            keepdims=True)
    pos2_ref[...] = jnp.sum(jnp.where(lanes == i2, dest, 0), axis=-1,
                            keepdims=True)

    jt = jax.lax.broadcasted_iota(jnp.int32, (NT, E), 0)
    ctb = jnp.broadcast_to(ct, (NT, E))
    te = jnp.sum((jt >= ctb).astype(jnp.int32), axis=-1, keepdims=True)
    te_ref[...] = jnp.minimum(te, E - 1)
    total = ctb[:, E - 1:E]
    valid_ref[...] = (jt[:, :1] < total).astype(jnp.int32)


def _router(x2d, Wg, bg):
    return pl.pallas_call(
        _router_body,
        out_shape=[
            jax.ShapeDtypeStruct((N, 1), jnp.int32),
            jax.ShapeDtypeStruct((N, 1), jnp.int32),
            jax.ShapeDtypeStruct((N, 1), jnp.float32),
            jax.ShapeDtypeStruct((N, 1), jnp.float32),
            jax.ShapeDtypeStruct((NT, 1), jnp.int32),
            jax.ShapeDtypeStruct((NT, 1), jnp.int32),
            jax.ShapeDtypeStruct((N, CP), jnp.int32),
        ],
    )(x2d, Wg, bg.reshape(1, E))


def _dispatch_body(pos1_hbm, pos2_hbm, pa_hbm, pb_hbm, x_hbm,
                   xs_hbm, scl_hbm,
                   pos1_v, pos2_v, pa_v, pb_v, rid2_v, scl_v,
                   rows_a, rows_b, sem_a, sem_b):
    wid = lax.axis_index("s") * NC + lax.axis_index("c")
    base = wid * RPW
    pltpu.sync_copy(pos1_hbm, pos1_v)
    pltpu.sync_copy(pos2_hbm, pos2_v)
    pltpu.sync_copy(pa_hbm, pa_v)
    pltpu.sync_copy(pb_hbm, pb_v)

    zf = jnp.zeros((L,), jnp.float32)
    zi = jnp.zeros((L,), jnp.int32)
    for i in range(RPW // L):
        scl_v[pl.ds(i * L, L)] = zf
    for c in range(NCH):
        for j in range(GCH // L):
            rid2_v[c, pl.ds(j * L, L)] = zi

    def scan(i, carry):
        tok = lax.iota(jnp.int32, L) + i * L
        for pv, sv in ((pos1_v, pa_v), (pos2_v, pb_v)):
            pos = pv[pl.ds(i * L, L)]
            rel = pos - base
            m = jnp.logical_and(rel >= 0, rel < RPW)
            relc = jnp.where(m, rel, 0)
            plsc.store_scatter(rid2_v, [lax.div(relc, GCH), lax.rem(relc, GCH)],
                               tok, mask=m)
            plsc.store_scatter(scl_v, [relc], sv[pl.ds(i * L, L)], mask=m)
        return carry

    lax.fori_loop(0, N // L, scan, 0)

    pltpu.sync_copy(scl_v, scl_hbm.at[pl.ds(base, RPW)])
    rows = (rows_a, rows_b)
    sems = (sem_a, sem_b)
    g = {}
    g[0] = pltpu.async_copy(x_hbm.at[rid2_v.at[0]], rows[0], sems[0])
    g[1] = pltpu.async_copy(x_hbm.at[rid2_v.at[1]], rows[1], sems[1])
    for c in range(NCH):
        b = c & 1
        g[c].wait()
        st = pltpu.async_copy(rows[b],
                              xs_hbm.at[pl.ds(base + c * GCH, GCH)],
                              sems[b])
        st.wait()
        if c + 2 < NCH:
            g[c + 2] = pltpu.async_copy(x_hbm.at[rid2_v.at[c + 2]],
                                        rows[b], sems[b])


def _dispatch(pos1, pos2, pa, pb, x2d):
    mesh = plsc.VectorSubcoreMesh(core_axis_name="c", subcore_axis_name="s")
    return pl.kernel(
        _dispatch_body,
        out_type=[
            jax.ShapeDtypeStruct((PADROWS, CP), jnp.int32),
            jax.ShapeDtypeStruct((PADROWS,), jnp.float32),
        ],
        mesh=mesh,
        scratch_types=[
            pltpu.VMEM((N,), jnp.int32),
            pltpu.VMEM((N,), jnp.int32),
            pltpu.VMEM((N,), jnp.float32),
            pltpu.VMEM((N,), jnp.float32),
            pltpu.VMEM((NCH, GCH), jnp.int32),
            pltpu.VMEM((RPW,), jnp.float32),
            pltpu.VMEM((GCH, CP), jnp.int32),
            pltpu.VMEM((GCH, CP), jnp.int32),
            pltpu.SemaphoreType.DMA,
            pltpu.SemaphoreType.DMA,
        ],
        compiler_params=pltpu.CompilerParams(needs_layout_passes=False),
    )(pos1, pos2, pa, pb, x2d)


def _gffn_body(te_ref, valid_ref, xs_ref, w1_ref, b1_ref, w2_ref, b2_ref,
               scl_ref, out_ref, acc, xbf):
    t = pl.program_id(0)
    h = pl.program_id(1)

    @pl.when(valid_ref[t] == 1)
    def _():
        @pl.when(h == 0)
        def _():
            w = xs_ref[...]
            lo = jax.lax.bitcast_convert_type(
                jax.lax.shift_left(w, 16), jnp.float32)
            hi = jax.lax.bitcast_convert_type(
                jnp.bitwise_and(w, jnp.int32(-65536)), jnp.float32)
            xbf[...] = jnp.concatenate([lo, hi], axis=1).astype(jnp.bfloat16)
            acc[...] = jnp.zeros((TILE, C), jnp.float32) + b2_ref[0]

        hh = jax.lax.dot_general(
            xbf[...], w1_ref[0], (((1,), (0,)), ((), ())),
            preferred_element_type=jnp.float32,
        ) + b1_ref[0]
        hbf = jnp.maximum(hh, 0.0).astype(jnp.bfloat16)
        acc[...] += jax.lax.dot_general(
            hbf, w2_ref[0], (((1,), (0,)), ((), ())),
            preferred_element_type=jnp.float32,
        )

        @pl.when(h == NH - 1)
        def _():
            rb = ((acc[...] * scl_ref[...]).astype(jnp.bfloat16)
                  .astype(jnp.float32))
            blo = jax.lax.bitcast_convert_type(rb[:, :CP], jnp.int32)
            bhi = jax.lax.bitcast_convert_type(rb[:, CP:], jnp.int32)
            out_ref[...] = jnp.bitwise_or(
                jax.lax.shift_right_logical(blo, 16),
                jnp.bitwise_and(bhi, jnp.int32(-65536)))


def _grouped_ffn(te, valid, xs, W1bf, b1r, W2bf, b2r, scl):
    def hh_of(t, h):
        return jnp.where(t % 2 == 1, NH - 1 - h, h)

    grid_spec = pltpu.PrefetchScalarGridSpec(
        num_scalar_prefetch=2,
        grid=(NT, NH),
        in_specs=[
            pl.BlockSpec((TILE, CP), lambda t, h, te, va: (t, 0)),
            pl.BlockSpec((1, C, HCG), lambda t, h, te, va: (te[t], 0, hh_of(t, h))),
            pl.BlockSpec((1, 1, HCG), lambda t, h, te, va: (te[t], 0, hh_of(t, h))),
            pl.BlockSpec((1, HCG, C), lambda t, h, te, va: (te[t], hh_of(t, h), 0)),
            pl.BlockSpec((1, 1, C), lambda t, h, te, va: (te[t], 0, 0)),
            pl.BlockSpec((TILE, 1), lambda t, h, te, va: (t, 0)),
        ],
        out_specs=pl.BlockSpec((TILE, CP), lambda t, h, te, va: (t, 0)),
        scratch_shapes=[pltpu.VMEM((TILE, C), jnp.float32),
                        pltpu.VMEM((TILE, C), jnp.bfloat16)],
    )
    return pl.pallas_call(
        _gffn_body,
        grid_spec=grid_spec,
        out_shape=jax.ShapeDtypeStruct((PADROWS, CP), jnp.int32),
        compiler_params=pltpu.CompilerParams(
            dimension_semantics=("arbitrary", "arbitrary"),
        ),
    )(te, valid, xs, W1bf, b1r, W2bf, b2r, scl)


def _combine_body(pos1_hbm, pos2_hbm, ys_hbm, out_hbm,
                  p1b, p2b, rows_a, rows_b, sem):
    wid = lax.axis_index("s") * NC + lax.axis_index("c")
    tbase = wid * TPW
    for c in range(TPW // CW):
        pltpu.sync_copy(pos1_hbm.at[pl.ds(tbase + c * CW, CW)], p1b.at[c])
        pltpu.sync_copy(pos2_hbm.at[pl.ds(tbase + c * CW, CW)], p2b.at[c])
        pltpu.async_copy(ys_hbm.at[p1b.at[c]], rows_a, sem).wait()
        pltpu.async_copy(ys_hbm.at[p2b.at[c]], rows_b, sem).wait()

        def add_chunk(l, carry):
            lo = l * L
            for r in range(CW):
                a = plsc.bitcast(rows_a[r, pl.ds(lo, L)], jnp.bfloat16)
                b = plsc.bitcast(rows_b[r, pl.ds(lo, L)], jnp.bfloat16)
                rows_a[r, pl.ds(lo, L)] = plsc.bitcast(a + b, jnp.int32)
            return carry

        lax.fori_loop(0, CP // L, add_chunk, 0)
        pltpu.sync_copy(rows_a, out_hbm.at[pl.ds(tbase + c * CW, CW)])


def _combine(pos1, pos2, ys):
    mesh = plsc.VectorSubcoreMesh(core_axis_name="c", subcore_axis_name="s")
    return pl.kernel(
        _combine_body,
        out_type=jax.ShapeDtypeStruct((N, CP), jnp.int32),
        mesh=mesh,
        scratch_types=[
            pltpu.VMEM((TPW // CW, CW), jnp.int32),
            pltpu.VMEM((TPW // CW, CW), jnp.int32),
            pltpu.VMEM((CW, CP), jnp.int32),
            pltpu.VMEM((CW, CP), jnp.int32),
            pltpu.SemaphoreType.DMA,
        ],
        compiler_params=pltpu.CompilerParams(needs_layout_passes=False),
    )(pos1, pos2, ys)


@jax.jit
def kernel(x, Wg, bg, W1, b1, W2, b2):
    x2d = x.reshape(N, C)
    pos1, pos2, p1, p2, te, valid, xp = _router(x2d, Wg, bg)
    xs32, scl = _dispatch(pos1.reshape(N), pos2.reshape(N),
                          p1.reshape(N), p2.reshape(N), xp)
    ys32 = _grouped_ffn(te.reshape(NT), valid.reshape(NT), xs32,
                        W1.astype(jnp.bfloat16), b1.reshape(E, 1, H),
                        W2.astype(jnp.bfloat16), b2.reshape(E, 1, C),
                        scl.reshape(PADROWS, 1))
    out32 = _combine(pos1.reshape(N), pos2.reshape(N), ys32)
    lo = jax.lax.bitcast_convert_type(
        jax.lax.shift_left(out32, 16), jnp.float32)
    hi = jax.lax.bitcast_convert_type(
        jnp.bitwise_and(out32, jnp.int32(-65536)), jnp.float32)
    out2d = jnp.concatenate([lo, hi], axis=1)
    return out2d.reshape(B, T, C)


# speedup vs baseline: 1.2881x; 1.0493x over previous
import functools

import jax
import jax.numpy as jnp
from jax import lax
from jax.experimental import pallas as pl
from jax.experimental.pallas import tpu as pltpu
from jax.experimental.pallas import tpu_sc as plsc

B, T, C, E, H, K = 2, 2048, 1024, 8, 4096, 2
N = B * T
TILE = 128
NT = 72
PADROWS = NT * TILE
HCG = 4096
NH = H // HCG

NC, NS, L = 2, 16, 16
NW = NC * NS
RPW = PADROWS // NW
GCH = 96
NCH = RPW // GCH
TPW = N // NW
CW = 32
CP = C // 2


def _cumsum0(a):
    n = a.shape[0]
    d = 1
    while d < n:
        a = a + jnp.concatenate(
            [jnp.zeros((d, a.shape[1]), a.dtype), a[: n - d]], axis=0)
        d *= 2
    return a


def _router_body(x_ref, wg_ref, bg_ref,
                 pos1_ref, pos2_ref, p1_ref, p2_ref, te_ref, valid_ref,
                 xp_ref):
    xb = x_ref[...].astype(jnp.bfloat16)
    rb = xb.astype(jnp.float32)
    blo = jax.lax.bitcast_convert_type(rb[:, :CP], jnp.int32)
    bhi = jax.lax.bitcast_convert_type(rb[:, CP:], jnp.int32)
    xp_ref[...] = jnp.bitwise_or(
        jax.lax.shift_right_logical(blo, 16),
        jnp.bitwise_and(bhi, jnp.int32(-65536)))
    z = jax.lax.dot_general(
        xb, wg_ref[...].astype(jnp.bfloat16),
        (((1,), (0,)), ((), ())),
        preferred_element_type=jnp.float32,
    ) + bg_ref[...]
    zmax = jnp.max(z, axis=-1, keepdims=True)
    ez = jnp.exp(z - zmax)
    s = ez / jnp.sum(ez, axis=-1, keepdims=True)
    lanes = jax.lax.broadcasted_iota(jnp.int32, (N, E), 1)
    m1 = jnp.max(s, axis=-1, keepdims=True)
    i1 = jnp.min(jnp.where(s == m1, lanes, E), axis=-1, keepdims=True)
    sm = jnp.where(lanes == i1, -jnp.inf, s)
    m2 = jnp.max(sm, axis=-1, keepdims=True)
    i2 = jnp.min(jnp.where(sm == m2, lanes, E), axis=-1, keepdims=True)
    e2 = jnp.exp(m2 - m1)
    p1_ref[...] = 1.0 / (1.0 + e2)
    p2_ref[...] = e2 / (1.0 + e2)

    flag = (jnp.equal(lanes, i1) | jnp.equal(lanes, i2)).astype(jnp.int32)
    cum = _cumsum0(flag)
    counts = cum[N - 1:N, :]
    tiles_e = (counts + (TILE - 1)) // TILE
    ct = tiles_e
    d = 1
    while d < E:
        ct = ct + jnp.concatenate(
            [jnp.zeros((1, d), ct.dtype), ct[:, : E - d]], axis=1)
        d *= 2
    base = (ct - tiles_e) * TILE
    dest = base + cum - 1
    pos1_ref[...] = jnp.sum(jnp.where(lanes == i1, dest, 0), axis=-1,
                            keepdims=True)
    pos2_ref[...] = jnp.sum(jnp.where(lanes == i2, dest, 0), axis=-1,
                            keepdims=True)

    jt = jax.lax.broadcasted_iota(jnp.int32, (NT, E), 0)
    ctb = jnp.broadcast_to(ct, (NT, E))
    te = jnp.sum((jt >= ctb).astype(jnp.int32), axis=-1, keepdims=True)
    te_ref[...] = jnp.minimum(te, E - 1)
    total = ctb[:, E - 1:E]
    valid_ref[...] = (jt[:, :1] < total).astype(jnp.int32)


def _router(x2d, Wg, bg):
    return pl.pallas_call(
        _router_body,
        out_shape=[
            jax.ShapeDtypeStruct((N, 1), jnp.int32),
            jax.ShapeDtypeStruct((N, 1), jnp.int32),
            jax.ShapeDtypeStruct((N, 1), jnp.float32),
            jax.ShapeDtypeStruct((N, 1), jnp.float32),
            jax.ShapeDtypeStruct((NT, 1), jnp.int32),
            jax.ShapeDtypeStruct((NT, 1), jnp.int32),
            jax.ShapeDtypeStruct((N, CP), jnp.int32),
        ],
    )(x2d, Wg, bg.reshape(1, E))


def _dispatch_body(pos1_hbm, pos2_hbm, pa_hbm, pb_hbm, x_hbm,
                   xs_hbm, scl_hbm,
                   pos1_v, pos2_v, pa_v, pb_v, rid2_v, scl_v,
                   rows_a, rows_b, sem_a, sem_b):
    wid = lax.axis_index("s") * NC + lax.axis_index("c")
    base = wid * RPW
    pltpu.sync_copy(pos1_hbm, pos1_v)
    pltpu.sync_copy(pos2_hbm, pos2_v)
    pltpu.sync_copy(pa_hbm, pa_v)
    pltpu.sync_copy(pb_hbm, pb_v)

    zf = jnp.zeros((L,), jnp.float32)
    zi = jnp.zeros((L,), jnp.int32)
    for i in range(RPW // L):
        scl_v[pl.ds(i * L, L)] = zf
    for c in range(NCH):
        for j in range(GCH // L):
            rid2_v[c, pl.ds(j * L, L)] = zi

    def scan(i, carry):
        tok = lax.iota(jnp.int32, L) + i * L
        for pv, sv in ((pos1_v, pa_v), (pos2_v, pb_v)):
            pos = pv[pl.ds(i * L, L)]
            rel = pos - base
            m = jnp.logical_and(rel >= 0, rel < RPW)
            relc = jnp.where(m, rel, 0)
            plsc.store_scatter(rid2_v, [lax.div(relc, GCH), lax.rem(relc, GCH)],
                               tok, mask=m)
            plsc.store_scatter(scl_v, [relc], sv[pl.ds(i * L, L)], mask=m)
        return carry

    lax.fori_loop(0, N // L, scan, 0)

    pltpu.sync_copy(scl_v, scl_hbm.at[pl.ds(base, RPW)])
    rows = (rows_a, rows_b)
    sems = (sem_a, sem_b)
    g = {}
    g[0] = pltpu.async_copy(x_hbm.at[rid2_v.at[0]], rows[0], sems[0])
    g[1] = pltpu.async_copy(x_hbm.at[rid2_v.at[1]], rows[1], sems[1])
    for c in range(NCH):
        b = c & 1
        g[c].wait()
        st = pltpu.async_copy(rows[b],
                              xs_hbm.at[pl.ds(base + c * GCH, GCH)],
                              sems[b])
        st.wait()
        if c + 2 < NCH:
            g[c + 2] = pltpu.async_copy(x_hbm.at[rid2_v.at[c + 2]],
                                        rows[b], sems[b])


def _dispatch(pos1, pos2, pa, pb, x2d):
    mesh = plsc.VectorSubcoreMesh(core_axis_name="c", subcore_axis_name="s")
    return pl.kernel(
        _dispatch_body,
        out_type=[
            jax.ShapeDtypeStruct((PADROWS, CP), jnp.int32),
            jax.ShapeDtypeStruct((PADROWS,), jnp.float32),
        ],
        mesh=mesh,
        scratch_types=[
            pltpu.VMEM((N,), jnp.int32),
            pltpu.VMEM((N,), jnp.int32),
            pltpu.VMEM((N,), jnp.float32),
            pltpu.VMEM((N,), jnp.float32),
            pltpu.VMEM((NCH, GCH), jnp.int32),
            pltpu.VMEM((RPW,), jnp.float32),
            pltpu.VMEM((GCH, CP), jnp.int32),
            pltpu.VMEM((GCH, CP), jnp.int32),
            pltpu.SemaphoreType.DMA,
            pltpu.SemaphoreType.DMA,
        ],
        compiler_params=pltpu.CompilerParams(needs_layout_passes=False),
    )(pos1, pos2, pa, pb, x2d)


def _gffn_body(te_ref, valid_ref, xs_ref, w1_ref, b1_ref, w2_ref, b2_ref,
               scl_ref, out_ref, acc, xbf):
    t = pl.program_id(0)
    h = pl.program_id(1)

    @pl.when(valid_ref[t] == 1)
    def _():
        @pl.when(h == 0)
        def _():
            w = xs_ref[...]
            lo = jax.lax.bitcast_convert_type(
                jax.lax.shift_left(w, 16), jnp.float32)
            hi = jax.lax.bitcast_convert_type(
                jnp.bitwise_and(w, jnp.int32(-65536)), jnp.float32)
            xbf[...] = jnp.concatenate([lo, hi], axis=1).astype(jnp.bfloat16)
            acc[...] = jnp.zeros((TILE, C), jnp.float32) + b2_ref[0]

        hh = jax.lax.dot_general(
            xbf[...], w1_ref[0], (((1,), (0,)), ((), ())),
            preferred_element_type=jnp.float32,
        ) + b1_ref[0]
        hbf = jnp.maximum(hh, 0.0).astype(jnp.bfloat16)
        acc[...] += jax.lax.dot_general(
            hbf, w2_ref[0], (((1,), (0,)), ((), ())),
            preferred_element_type=jnp.float32,
        )

        @pl.when(h == NH - 1)
        def _():
            rb = ((acc[...] * scl_ref[...]).astype(jnp.bfloat16)
                  .astype(jnp.float32))
            blo = jax.lax.bitcast_convert_type(rb[:, :CP], jnp.int32)
            bhi = jax.lax.bitcast_convert_type(rb[:, CP:], jnp.int32)
            out_ref[...] = jnp.bitwise_or(
                jax.lax.shift_right_logical(blo, 16),
                jnp.bitwise_and(bhi, jnp.int32(-65536)))


def _grouped_ffn(te, valid, xs, W1bf, b1r, W2bf, b2r, scl):
    def hh_of(t, h):
        return jnp.where(t % 2 == 1, NH - 1 - h, h)

    grid_spec = pltpu.PrefetchScalarGridSpec(
        num_scalar_prefetch=2,
        grid=(NT, NH),
        in_specs=[
            pl.BlockSpec((TILE, CP), lambda t, h, te, va: (t, 0)),
            pl.BlockSpec((1, C, HCG), lambda t, h, te, va: (te[t], 0, hh_of(t, h))),
            pl.BlockSpec((1, 1, HCG), lambda t, h, te, va: (te[t], 0, hh_of(t, h))),
            pl.BlockSpec((1, HCG, C), lambda t, h, te, va: (te[t], hh_of(t, h), 0)),
            pl.BlockSpec((1, 1, C), lambda t, h, te, va: (te[t], 0, 0)),
            pl.BlockSpec((TILE, 1), lambda t, h, te, va: (t, 0)),
        ],
        out_specs=pl.BlockSpec((TILE, CP), lambda t, h, te, va: (t, 0)),
        scratch_shapes=[pltpu.VMEM((TILE, C), jnp.float32),
                        pltpu.VMEM((TILE, C), jnp.bfloat16)],
    )
    return pl.pallas_call(
        _gffn_body,
        grid_spec=grid_spec,
        out_shape=jax.ShapeDtypeStruct((PADROWS, CP), jnp.int32),
        compiler_params=pltpu.CompilerParams(
            dimension_semantics=("arbitrary", "arbitrary"),
        ),
    )(te, valid, xs, W1bf, b1r, W2bf, b2r, scl)


def _combine_body(pos1_hbm, pos2_hbm, ys_hbm, out_hbm,
                  p1b, p2b, rows_a, rows_b, sem):
    wid = lax.axis_index("s") * NC + lax.axis_index("c")
    tbase = wid * TPW
    for c in range(TPW // CW):
        pltpu.sync_copy(pos1_hbm.at[pl.ds(tbase + c * CW, CW)], p1b.at[c])
        pltpu.sync_copy(pos2_hbm.at[pl.ds(tbase + c * CW, CW)], p2b.at[c])
        pltpu.async_copy(ys_hbm.at[p1b.at[c]], rows_a, sem).wait()
        pltpu.async_copy(ys_hbm.at[p2b.at[c]], rows_b, sem).wait()

        def add_chunk(l, carry):
            lo = l * L
            for r in range(CW):
                a = plsc.bitcast(rows_a[r, pl.ds(lo, L)], jnp.bfloat16)
                b = plsc.bitcast(rows_b[r, pl.ds(lo, L)], jnp.bfloat16)
                rows_a[r, pl.ds(lo, L)] = plsc.bitcast(a + b, jnp.int32)
            return carry

        lax.fori_loop(0, CP // L, add_chunk, 0)
        pltpu.sync_copy(rows_a, out_hbm.at[pl.ds(tbase + c * CW, CW)])


def _combine(pos1, pos2, ys):
    mesh = plsc.VectorSubcoreMesh(core_axis_name="c", subcore_axis_name="s")
    return pl.kernel(
        _combine_body,
        out_type=jax.ShapeDtypeStruct((N, CP), jnp.int32),
        mesh=mesh,
        scratch_types=[
            pltpu.VMEM((TPW // CW, CW), jnp.int32),
            pltpu.VMEM((TPW // CW, CW), jnp.int32),
            pltpu.VMEM((CW, CP), jnp.int32),
            pltpu.VMEM((CW, CP), jnp.int32),
            pltpu.SemaphoreType.DMA,
        ],
        compiler_params=pltpu.CompilerParams(needs_layout_passes=False),
    )(pos1, pos2, ys)


@jax.jit
def kernel(x, Wg, bg, W1, b1, W2, b2):
    x2d = x.reshape(N, C)
    pos1, pos2, p1, p2, te, valid, xp = _router(x2d, Wg, bg)
    xs32, scl = _dispatch(pos1.reshape(N), pos2.reshape(N),
                          p1.reshape(N), p2.reshape(N), xp)
    ys32 = _grouped_ffn(te.reshape(NT), valid.reshape(NT), xs32,
                        W1.astype(jnp.bfloat16), b1.reshape(E, 1, H),
                        W2.astype(jnp.bfloat16), b2.reshape(E, 1, C),
                        scl.reshape(PADROWS, 1))
    out32 = _combine(pos1.reshape(N), pos2.reshape(N), ys32)
    lo = jax.lax.bitcast_convert_type(
        jax.lax.shift_left(out32, 16), jnp.float32)
    hi = jax.lax.bitcast_convert_type(
        jnp.bitwise_and(out32, jnp.int32(-65536)), jnp.float32)
    out2d = jnp.concatenate([lo, hi], axis=1)
    return out2d.reshape(B, T, C)
